# Initial kernel scaffold; baseline (speedup 1.0000x reference)
import jax
import jax.numpy as jnp
from jax.experimental import pallas as pl


def kernel(x, table):
    raise NotImplementedError("write your pallas kernel here")


# speedup vs baseline: 4.0625x; 4.0625x over previous
import functools

import jax
import jax.numpy as jnp
from jax import lax
from jax.experimental import pallas as pl
from jax.experimental.pallas import tpu as pltpu
from jax.experimental.pallas import tpu_sc as plsc

B, L, D = 4096, 200, 64
TOT = B * L
NW = 32
BPW = TOT // NW
C = 512
NCHUNK = BPW // C

_MESH = plsc.VectorSubcoreMesh(core_axis_name="c", subcore_axis_name="s")


@functools.partial(
    pl.kernel,
    mesh=_MESH,
    out_type=jax.ShapeDtypeStruct((TOT, D), jnp.float32),
    scratch_types=[
        pltpu.VMEM((C,), jnp.int32),
        pltpu.VMEM((C, D), jnp.float32),
        pltpu.SemaphoreType.DMA,
    ],
    compiler_params=pltpu.CompilerParams(use_tc_tiling_on_sc=False),
)
def _emb_gather(idx_hbm, table_hbm, out_hbm, idx_v, rows_v, sem):
    wid = lax.axis_index("s") * 2 + lax.axis_index("c")
    base = wid * BPW

    def body(ci, _):
        off = base + ci * C
        pltpu.sync_copy(idx_hbm.at[pl.ds(off, C)], idx_v)
        pltpu.async_copy(table_hbm.at[idx_v], rows_v, sem).wait()
        pltpu.sync_copy(rows_v, out_hbm.at[pl.ds(off, C)])
        return 0

    lax.fori_loop(0, NCHUNK, body, 0)


def kernel(x, table):
    flat = x.reshape(TOT).astype(jnp.int32)
    out = _emb_gather(flat, table)
    return out.reshape(B, L, D)


# speedup vs baseline: 4.3507x; 1.0709x over previous
import functools

import jax
import jax.numpy as jnp
from jax import lax
from jax.experimental import pallas as pl
from jax.experimental.pallas import tpu as pltpu
from jax.experimental.pallas import tpu_sc as plsc

B, L, D = 4096, 200, 64
TOT = B * L
NW = 32
BPW = TOT // NW
C = 512
NCHUNK = BPW // C

_MESH = plsc.VectorSubcoreMesh(core_axis_name="c", subcore_axis_name="s")


@functools.partial(
    pl.kernel,
    mesh=_MESH,
    out_type=jax.ShapeDtypeStruct((TOT, D), jnp.float32),
    scratch_types=[
        pltpu.VMEM((BPW,), jnp.int32),
        pltpu.VMEM((C, D), jnp.float32),
        pltpu.VMEM((C, D), jnp.float32),
        pltpu.SemaphoreType.DMA,
        pltpu.SemaphoreType.DMA,
        pltpu.SemaphoreType.DMA,
        pltpu.SemaphoreType.DMA,
    ],
    compiler_params=pltpu.CompilerParams(use_tc_tiling_on_sc=False),
)
def _emb_gather(idx_hbm, table_hbm, out_hbm, idx_all, rows0, rows1,
                g0, g1, o0, o1):
    wid = lax.axis_index("s") * 2 + lax.axis_index("c")
    base = wid * BPW
    pltpu.sync_copy(idx_hbm.at[pl.ds(base, BPW)], idx_all)

    def gather(ci, rows, g):
        pltpu.async_copy(table_hbm.at[idx_all.at[pl.ds(ci * C, C)]], rows, g)

    def store(ci, rows, o):
        pltpu.async_copy(rows, out_hbm.at[pl.ds(base + ci * C, C)], o)

    def wait_gather(rows, g):
        pltpu.make_async_copy(table_hbm.at[idx_all.at[pl.ds(0, C)]], rows, g).wait()

    def wait_store(rows, o):
        pltpu.make_async_copy(rows, out_hbm.at[pl.ds(base, C)], o).wait()

    gather(0, rows0, g0)
    gather(1, rows1, g1)
    wait_gather(rows0, g0)
    store(0, rows0, o0)
    wait_gather(rows1, g1)
    store(1, rows1, o1)

    def body(k, _):
        c0 = 2 * k
        wait_store(rows0, o0)
        gather(c0, rows0, g0)
        wait_store(rows1, o1)
        gather(c0 + 1, rows1, g1)
        wait_gather(rows0, g0)
        store(c0, rows0, o0)
        wait_gather(rows1, g1)
        store(c0 + 1, rows1, o1)
        return 0

    lax.fori_loop(1, NCHUNK // 2, body, 0)
    wait_store(rows0, o0)
    wait_store(rows1, o1)


def kernel(x, table):
    flat = x.reshape(TOT).astype(jnp.int32)
    out = _emb_gather(flat, table)
    return out.reshape(B, L, D)
